# baseline (speedup 1.0000x reference)
import functools

import jax
import jax.numpy as jnp
import numpy as np
from jax import lax
from jax.experimental import pallas as pl
from jax.experimental.pallas import tpu as pltpu
from jax.experimental.pallas import tpu_sc as plsc

N = 10000
E = 320000
D = 128
NPAD = 10240
EPAD = 327680
NC = 2
NS = 16
NW = NC * NS
CHUNK = 128
CHUNKS = EPAD // (NW * CHUNK)
SUPER = 40
KBUF = 2
STRIPE = NPAD // NS
DEGW = 16


def _sc_agg_body(h_hbm, src_hbm, dst_hbm, acc_out,
                 srcv, dstv, rows, acc_sh, *sems):
    cid = lax.axis_index("c")
    sid = lax.axis_index("s")
    w = cid * NS + sid
    row0 = sid * STRIPE

    zero = jnp.zeros((16,), jnp.float32)
    for r in range(CHUNK):
        for c in range(D // 16):
            rows[0, r, pl.ds(c * 16, 16)] = zero
    for k in range(STRIPE // CHUNK):
        pltpu.sync_copy(rows.at[0], acc_sh.at[pl.ds(row0 + k * CHUNK, CHUNK)])
    plsc.subcore_barrier()

    gsem0, gsem1, ssem0, ssem1 = sems

    @pl.loop(0, CHUNKS // SUPER)
    def _super_loop(ss):
        pltpu.sync_copy(src_hbm.at[w, pl.ds(ss * SUPER, SUPER)], srcv)
        pltpu.sync_copy(dst_hbm.at[w, pl.ds(ss * SUPER, SUPER)], dstv)
        pltpu.async_copy(h_hbm.at[srcv.at[0]], rows.at[0], gsem0)

        @pl.loop(0, SUPER // 2)
        def _pair_loop(i):
            a = 2 * i
            b = a + 1

            @pl.when(i > 0)
            def _drain_prev_odd():
                pltpu.make_async_copy(rows.at[1], acc_sh.at[dstv.at[a - 1]],
                                      ssem1).wait()

            pltpu.async_copy(h_hbm.at[srcv.at[b]], rows.at[1], gsem1)
            pltpu.make_async_copy(h_hbm.at[srcv.at[a]], rows.at[0],
                                  gsem0).wait()
            pltpu.async_copy(rows.at[0], acc_sh.at[dstv.at[a]], ssem0,
                             add=True)
            pltpu.make_async_copy(rows.at[0], acc_sh.at[dstv.at[a]],
                                  ssem0).wait()

            @pl.when(i < SUPER // 2 - 1)
            def _next_even_gather():
                pltpu.async_copy(h_hbm.at[srcv.at[a + 2]], rows.at[0], gsem0)

            pltpu.make_async_copy(h_hbm.at[srcv.at[b]], rows.at[1],
                                  gsem1).wait()
            pltpu.async_copy(rows.at[1], acc_sh.at[dstv.at[b]], ssem1,
                             add=True)

        pltpu.make_async_copy(rows.at[1], acc_sh.at[dstv.at[SUPER - 1]],
                              ssem1).wait()

    plsc.subcore_barrier()
    pltpu.sync_copy(acc_sh.at[pl.ds(row0, STRIPE)],
                    acc_out.at[cid, pl.ds(row0, STRIPE)])


def _make_sc_aggregate():
    mesh = plsc.VectorSubcoreMesh(core_axis_name="c", subcore_axis_name="s",
                                  num_cores=NC, num_subcores=NS)
    return pl.kernel(
        _sc_agg_body,
        out_type=jax.ShapeDtypeStruct((NC, NPAD, D), jnp.float32),
        mesh=mesh,
        compiler_params=pltpu.CompilerParams(use_tc_tiling_on_sc=False),
        scratch_types=[
            pltpu.VMEM((SUPER, CHUNK), jnp.int32),
            pltpu.VMEM((SUPER, CHUNK), jnp.int32),
            pltpu.VMEM((KBUF, CHUNK, D), jnp.float32),
            pltpu.VMEM_SHARED((NPAD, D), jnp.float32),
        ] + [pltpu.SemaphoreType.DMA] * 4,
    )


def _sc_deg_body(dst_hbm, deg_out, dstv, onesv, zv, deg_sh):
    cid = lax.axis_index("c")
    sid = lax.axis_index("s")
    w = cid * NS + sid
    row0 = sid * STRIPE

    zero = jnp.zeros((16,), jnp.float32)
    one = jnp.ones((16,), jnp.float32)
    for i in range(CHUNK):
        zv[i, :] = zero
        onesv[i, :] = one
    for k in range(STRIPE // CHUNK):
        pltpu.sync_copy(zv, deg_sh.at[pl.ds(row0 + k * CHUNK, CHUNK)])
    pltpu.sync_copy(dst_hbm.at[w], dstv)
    plsc.subcore_barrier()

    @pl.loop(0, CHUNKS)
    def _chunk_loop(g):
        pltpu.sync_copy(onesv, deg_sh.at[dstv.at[g]], add=True)

    plsc.subcore_barrier()
    pltpu.sync_copy(deg_sh.at[pl.ds(row0, STRIPE)],
                    deg_out.at[cid, pl.ds(row0, STRIPE)])


def _make_sc_degree():
    mesh = plsc.VectorSubcoreMesh(core_axis_name="c", subcore_axis_name="s",
                                  num_cores=NC, num_subcores=NS)
    return pl.kernel(
        _sc_deg_body,
        out_type=jax.ShapeDtypeStruct((NC, NPAD, DEGW), jnp.float32),
        mesh=mesh,
        compiler_params=pltpu.CompilerParams(use_tc_tiling_on_sc=False),
        scratch_types=[
            pltpu.VMEM((CHUNKS, CHUNK), jnp.int32),
            pltpu.VMEM((CHUNK, DEGW), jnp.float32),
            pltpu.VMEM((CHUNK, DEGW), jnp.float32),
            pltpu.VMEM_SHARED((NPAD, DEGW), jnp.float32),
        ],
    )


_TC_BLK = 2048


def _mm_bias_body(x_ref, w_ref, b_ref, o_ref):
    o_ref[...] = jnp.dot(x_ref[...], w_ref[...],
                         preferred_element_type=jnp.float32) + b_ref[...]


def _tc_in_proj(x, W_in, b_in):
    return pl.pallas_call(
        _mm_bias_body,
        grid=(NPAD // _TC_BLK,),
        in_specs=[
            pl.BlockSpec((_TC_BLK, D), lambda i: (i, 0)),
            pl.BlockSpec((D, D), lambda i: (0, 0)),
            pl.BlockSpec((1, D), lambda i: (0, 0)),
        ],
        out_specs=pl.BlockSpec((_TC_BLK, D), lambda i: (i, 0)),
        out_shape=jax.ShapeDtypeStruct((NPAD, D), jnp.float32),
    )(x, W_in, b_in.reshape(1, D))


def _combine_body(final, acc_ref, deg_ref, h_ref, wn_ref, ws_ref, b_ref,
                  *rest):
    if final:
        wo_ref, bo_ref, o_ref = rest
    else:
        (o_ref,) = rest
    s = acc_ref[0] + acc_ref[1]
    dsum = deg_ref[0, :, :1] + deg_ref[1, :, :1]
    inv = 1.0 / jnp.maximum(dsum, 1.0)
    agg = s * inv
    z = (jnp.dot(agg, wn_ref[...], preferred_element_type=jnp.float32)
         + jnp.dot(h_ref[...], ws_ref[...], preferred_element_type=jnp.float32)
         + b_ref[...])
    z = jnp.maximum(z, 0.0)
    if final:
        z = jnp.dot(z, wo_ref[...],
                    preferred_element_type=jnp.float32) + bo_ref[...]
    o_ref[...] = z


def _tc_combine(acc, deg, h, Wn, Ws, b, final=None):
    in_specs = [
        pl.BlockSpec((NC, _TC_BLK, D), lambda i: (0, i, 0)),
        pl.BlockSpec((NC, _TC_BLK, DEGW), lambda i: (0, i, 0)),
        pl.BlockSpec((_TC_BLK, D), lambda i: (i, 0)),
        pl.BlockSpec((D, D), lambda i: (0, 0)),
        pl.BlockSpec((D, D), lambda i: (0, 0)),
        pl.BlockSpec((1, D), lambda i: (0, 0)),
    ]
    args = [acc, deg, h, Wn, Ws, b.reshape(1, D)]
    out_rows = NPAD
    if final is not None:
        Wo, bo = final
        in_specs.append(pl.BlockSpec((D, D), lambda i: (0, 0)))
        in_specs.append(pl.BlockSpec((1, D), lambda i: (0, 0)))
        args.extend([Wo, bo.reshape(1, D)])
        out_rows = N
    return pl.pallas_call(
        functools.partial(_combine_body, final is not None),
        grid=(NPAD // _TC_BLK,),
        in_specs=in_specs,
        out_specs=pl.BlockSpec((_TC_BLK, D), lambda i: (i, 0)),
        out_shape=jax.ShapeDtypeStruct((out_rows, D), jnp.float32),
    )(*args)


_PAD_IDX = np.asarray(N + (np.arange(EPAD - E) % (NPAD - N)), dtype=np.int32)


def kernel(x, edge_index, W_in, b_in, W_neigh, W_self, b_hidden, W_out, b_out):
    ei = jnp.concatenate(
        [edge_index.astype(jnp.int32),
         jnp.broadcast_to(_PAD_IDX, (2, EPAD - E))], axis=1
    ).reshape(2, NW, CHUNKS, CHUNK)
    dst_p = ei[0]
    src_p = ei[1]

    sc_agg = _make_sc_aggregate()
    sc_deg = _make_sc_degree()

    h0 = _tc_in_proj(x, W_in, b_in)
    deg = sc_deg(dst_p)
    acc0 = sc_agg(h0, src_p, dst_p)
    h1 = _tc_combine(acc0, deg, h0, W_neigh[0], W_self[0], b_hidden[0])
    acc1 = sc_agg(h1, src_p, dst_p)
    return _tc_combine(acc1, deg, h1, W_neigh[1], W_self[1], b_hidden[1],
                       final=(W_out, b_out))
